# Initial kernel scaffold; baseline (speedup 1.0000x reference)
import functools

import jax
import jax.numpy as jnp
import numpy as np
from jax import lax
from jax.experimental import pallas as pl
from jax.experimental.pallas import tpu as pltpu
from jax.experimental.pallas import tpu_sc as plsc

NC = 2
NS = 16
NW = NC * NS
L = 16

V = 10
D = 10
N_TOK = 4 * 8192
TOK_PER_W = N_TOK // NW
OUT_PER_W = TOK_PER_W * D
BLOCKS = TOK_PER_W // L

_w = np.arange(D * L).reshape(D, L)
_ROWPAT = (_w // D).astype(np.int32)
_COLPAT = (_w % D).astype(np.int32)


def _body(ids_hbm, par_hbm, pat_hbm, out_hbm, par_v, pat_v, table_v,
          ids_v, out_v):
    wid = lax.axis_index("s") * NC + lax.axis_index("c")
    pltpu.sync_copy(par_hbm, par_v)
    pltpu.sync_copy(pat_hbm, pat_v)
    pltpu.sync_copy(ids_hbm.at[pl.ds(wid * TOK_PER_W, TOK_PER_W)], ids_v)

    for v in range(V):
        vv = jnp.full((L,), v, jnp.int32)
        h = par_v[30, :]
        for k in range(D):
            kk = jnp.full((L,), k, jnp.int32)
            s = plsc.load_gather(par_v, [vv, kk])
            h = h + s * par_v[10 + k, :]
        table_v[v, :] = h
    for v in range(V):
        vv = jnp.full((L,), v, jnp.int32)
        t = par_v[31, :]
        for k in range(D):
            kk = jnp.full((L,), k, jnp.int32)
            s = plsc.load_gather(table_v, [vv, kk])
            t = t + s * par_v[20 + k, :]
        table_v[v, :] = t

    def blk(b, _):
        tok0 = b * L
        for v in range(V):
            sel = plsc.load_gather(ids_v, [tok0 + pat_v[v, :]])
            val = plsc.load_gather(table_v, [sel, pat_v[10 + v, :]])
            out_v[pl.ds(b * (D * L) + v * L, L)] = val
        return 0

    lax.fori_loop(0, BLOCKS, blk, 0)
    pltpu.sync_copy(out_v, out_hbm.at[pl.ds(wid * OUT_PER_W, OUT_PER_W)])


@functools.partial(
    pl.kernel,
    out_type=jax.ShapeDtypeStruct((N_TOK * D,), jnp.float32),
    mesh=plsc.VectorSubcoreMesh(core_axis_name="c", subcore_axis_name="s"),
    scratch_types=[
        pltpu.VMEM((32, L), jnp.float32),
        pltpu.VMEM((2 * V, L), jnp.int32),
        pltpu.VMEM((V, L), jnp.float32),
        pltpu.VMEM((TOK_PER_W,), jnp.int32),
        pltpu.VMEM((OUT_PER_W,), jnp.float32),
    ],
)
def _sc_lookup(ids_hbm, par_hbm, pat_hbm, out_hbm, *scratch):
    _body(ids_hbm, par_hbm, pat_hbm, out_hbm, *scratch)


def kernel(input_ids, emb, W1, b1, Wh, bh):
    ids = input_ids.reshape(-1).astype(jnp.int32)

    def pad16(x):
        return jnp.pad(x, ((0, 0), (0, L - D)))

    par = jnp.concatenate(
        [
            pad16(emb),
            pad16(W1.T),
            pad16(Wh.T),
            jnp.pad(b1, (0, L - D))[None, :],
            jnp.pad(bh, (0, L - D))[None, :],
        ],
        axis=0,
    )
    pat = jnp.asarray(np.concatenate([_ROWPAT, _COLPAT], axis=0))
    out = _sc_lookup(ids, par, pat)
    return out.reshape(4, 8192, D)


# speedup vs baseline: 1.7491x; 1.7491x over previous
import functools

import jax
import jax.numpy as jnp
import numpy as np
from jax import lax
from jax.experimental import pallas as pl
from jax.experimental.pallas import tpu as pltpu
from jax.experimental.pallas import tpu_sc as plsc

NC = 2
NS = 16
NW = NC * NS
L = 16

V = 10
D = 10
N_TOK = 4 * 8192
TOK_PER_W = N_TOK // NW
OUT_PER_W = TOK_PER_W * D
BLOCKS = TOK_PER_W // L

_w = np.arange(D * L).reshape(D, L)
_ROWPAT = (_w // D).astype(np.int32)
_COLPAT = (_w % D).astype(np.int32)


def _body(ids_hbm, par_hbm, pat_hbm, out_hbm, par_v, pat_v, table_v,
          ids_v, out_v):
    wid = lax.axis_index("s") * NC + lax.axis_index("c")
    pltpu.sync_copy(par_hbm, par_v)
    pltpu.sync_copy(pat_hbm, pat_v)
    pltpu.sync_copy(ids_hbm.at[pl.ds(wid * TOK_PER_W, TOK_PER_W)], ids_v)

    def par_row(r):
        return par_v[pl.ds(r * L, L)]

    def splat(x, k):
        return lax.gather(
            x,
            jnp.full((L, 1), k, jnp.int32),
            lax.GatherDimensionNumbers(
                offset_dims=(),
                collapsed_slice_dims=(0,),
                start_index_map=(0,),
            ),
            (1,),
            mode=lax.GatherScatterMode.PROMISE_IN_BOUNDS,
        )

    for v in range(V):
        e = par_row(v)
        h = par_row(30)
        for k in range(D):
            h = h + splat(e, k) * par_row(10 + k)
        t = par_row(31)
        for k in range(D):
            t = t + splat(h, k) * par_row(20 + k)
        table_v[pl.ds(v * L, L)] = t

    def blk(b, _):
        tok0 = b * L
        for v in range(V):
            sel = plsc.load_gather(ids_v, [tok0 + pat_v[pl.ds(v * L, L)]])
            flat = (sel << 4) + pat_v[pl.ds((V + v) * L, L)]
            val = plsc.load_gather(table_v, [flat])
            out_v[pl.ds(b * (D * L) + v * L, L)] = val
        return 0

    lax.fori_loop(0, BLOCKS, blk, 0)
    pltpu.sync_copy(out_v, out_hbm.at[pl.ds(wid * OUT_PER_W, OUT_PER_W)])


@functools.partial(
    pl.kernel,
    out_type=jax.ShapeDtypeStruct((N_TOK * D,), jnp.float32),
    mesh=plsc.VectorSubcoreMesh(core_axis_name="c", subcore_axis_name="s"),
    compiler_params=pltpu.CompilerParams(needs_layout_passes=False),
    scratch_types=[
        pltpu.VMEM((32 * L,), jnp.float32),
        pltpu.VMEM((2 * V * L,), jnp.int32),
        pltpu.VMEM((V * L,), jnp.float32),
        pltpu.VMEM((TOK_PER_W,), jnp.int32),
        pltpu.VMEM((OUT_PER_W,), jnp.float32),
    ],
)
def _sc_lookup(ids_hbm, par_hbm, pat_hbm, out_hbm, *scratch):
    _body(ids_hbm, par_hbm, pat_hbm, out_hbm, *scratch)


def kernel(input_ids, emb, W1, b1, Wh, bh):
    ids = input_ids.reshape(-1).astype(jnp.int32)

    def pad16(x):
        return jnp.pad(x, ((0, 0), (0, L - D)))

    par = jnp.concatenate(
        [
            pad16(emb),
            pad16(W1.T),
            pad16(Wh.T),
            jnp.pad(b1, (0, L - D))[None, :],
            jnp.pad(bh, (0, L - D))[None, :],
        ],
        axis=0,
    ).reshape(-1)
    pat = jnp.asarray(np.concatenate([_ROWPAT, _COLPAT], axis=0).reshape(-1))
    out = _sc_lookup(ids, par, pat)
    return out.reshape(4, 8192, D)


# speedup vs baseline: 1.9448x; 1.1119x over previous
import functools

import jax
import jax.numpy as jnp
import numpy as np
from jax import lax
from jax.experimental import pallas as pl
from jax.experimental.pallas import tpu as pltpu
from jax.experimental.pallas import tpu_sc as plsc

NC = 2
NS = 16
NW = NC * NS
L = 16

V = 10
D = 10
N_TOK = 4 * 8192
TOK_PER_W = N_TOK // NW
OUT_PER_W = TOK_PER_W * D
BLOCKS = TOK_PER_W // L

_w = np.arange(D * L).reshape(D, L)
_ROWPAT = (_w // D).astype(np.int32)
_COLPAT = (_w % D).astype(np.int32)

_GATHER_DNUMS = lax.GatherDimensionNumbers(
    offset_dims=(), collapsed_slice_dims=(0,), start_index_map=(0,))


def _reg_gather(x, idx16):
    return lax.gather(x, idx16.reshape(L, 1), _GATHER_DNUMS, (1,),
                      mode=lax.GatherScatterMode.PROMISE_IN_BOUNDS)


def _splat(x, k):
    return _reg_gather(x, jnp.full((L,), k, jnp.int32))


def _body(ids_hbm, par_hbm, pat_hbm, out_hbm, par_v, pat_v, table_v,
          ids_v, out_v):
    wid = lax.axis_index("s") * NC + lax.axis_index("c")
    pltpu.sync_copy(par_hbm, par_v)
    pltpu.sync_copy(pat_hbm, pat_v)
    pltpu.sync_copy(ids_hbm.at[pl.ds(wid * TOK_PER_W, TOK_PER_W)], ids_v)

    def par_row(r):
        return par_v[pl.ds(r * L, L)]

    for v in range(V):
        e = par_row(v)
        h = par_row(30)
        for k in range(D):
            h = h + _splat(e, k) * par_row(10 + k)
        t = par_row(31)
        for k in range(D):
            t = t + _splat(h, k) * par_row(20 + k)
        table_v[pl.ds(v * L, L)] = t

    rp = [pat_v[pl.ds(v * L, L)] for v in range(V)]
    cp = [pat_v[pl.ds((V + v) * L, L)] for v in range(V)]

    def blk(b, _):
        sid16 = ids_v[pl.ds(b * L, L)] << 4
        for v in range(V):
            flat = _reg_gather(sid16, rp[v]) + cp[v]
            out_v[pl.ds(b * (D * L) + v * L, L)] = plsc.load_gather(
                table_v, [flat])
        return 0

    lax.fori_loop(0, BLOCKS, blk, 0)
    pltpu.sync_copy(out_v, out_hbm.at[pl.ds(wid * OUT_PER_W, OUT_PER_W)])


@functools.partial(
    pl.kernel,
    out_type=jax.ShapeDtypeStruct((N_TOK * D,), jnp.float32),
    mesh=plsc.VectorSubcoreMesh(core_axis_name="c", subcore_axis_name="s"),
    compiler_params=pltpu.CompilerParams(needs_layout_passes=False),
    scratch_types=[
        pltpu.VMEM((32 * L,), jnp.float32),
        pltpu.VMEM((2 * V * L,), jnp.int32),
        pltpu.VMEM((V * L,), jnp.float32),
        pltpu.VMEM((TOK_PER_W,), jnp.int32),
        pltpu.VMEM((OUT_PER_W,), jnp.float32),
    ],
)
def _sc_lookup(ids_hbm, par_hbm, pat_hbm, out_hbm, *scratch):
    _body(ids_hbm, par_hbm, pat_hbm, out_hbm, *scratch)


def kernel(input_ids, emb, W1, b1, Wh, bh):
    ids = input_ids.reshape(-1).astype(jnp.int32)

    def pad16(x):
        return jnp.pad(x, ((0, 0), (0, L - D)))

    par = jnp.concatenate(
        [
            pad16(emb),
            pad16(W1.T),
            pad16(Wh.T),
            jnp.pad(b1, (0, L - D))[None, :],
            jnp.pad(bh, (0, L - D))[None, :],
        ],
        axis=0,
    ).reshape(-1)
    pat = jnp.asarray(np.concatenate([_ROWPAT, _COLPAT], axis=0).reshape(-1))
    return _sc_lookup(ids, par, pat).reshape(4, 8192, D)


# speedup vs baseline: 2.6403x; 1.3576x over previous
import functools

import jax
import jax.numpy as jnp
import numpy as np
from jax import lax
from jax.experimental import pallas as pl
from jax.experimental.pallas import tpu as pltpu
from jax.experimental.pallas import tpu_sc as plsc

NC = 2
NS = 16
NW = NC * NS
L = 16

V = 10
D = 10
N_TOK = 4 * 8192
TOK_PER_W = N_TOK // NW
CHUNK = 512

_GATHER_DNUMS = lax.GatherDimensionNumbers(
    offset_dims=(), collapsed_slice_dims=(0,), start_index_map=(0,))


def _reg_gather(x, idx16):
    return lax.gather(x, idx16.reshape(L, 1), _GATHER_DNUMS, (1,),
                      mode=lax.GatherScatterMode.PROMISE_IN_BOUNDS)


def _splat(x, k):
    return _reg_gather(x, jnp.full((L,), k, jnp.int32))


def _body(ids_hbm, par_hbm, out_hbm, par_v, table_v, ids_v, out_v):
    wid = lax.axis_index("s") * NC + lax.axis_index("c")
    pltpu.sync_copy(par_hbm, par_v)
    pltpu.sync_copy(ids_hbm.at[pl.ds(wid * TOK_PER_W, TOK_PER_W)], ids_v)

    def par_row(r):
        return par_v[pl.ds(r * L, L)]

    for v in range(V):
        e = par_row(v)
        h = par_row(30)
        for k in range(D):
            h = h + _splat(e, k) * par_row(10 + k)
        t = par_row(31)
        for k in range(D):
            t = t + _splat(h, k) * par_row(20 + k)
        table_v[pl.ds(v * L, L)] = t

    iota16 = lax.iota(jnp.int32, L)

    for c in range(TOK_PER_W // CHUNK):

        def blk(b, _):
            sid16 = ids_v[pl.ds(c * CHUNK + b * L, L)] << 4
            for j in range(L):
                idx = _reg_gather(sid16, jnp.full((L,), j, jnp.int32)) + iota16
                out_v[pl.ds((b * L + j) * 128, L)] = plsc.load_gather(
                    table_v, [idx])
            return 0

        lax.fori_loop(0, CHUNK // L, blk, 0)
        pltpu.sync_copy(
            out_v,
            out_hbm.at[pl.ds((wid * TOK_PER_W + c * CHUNK) * 128,
                             CHUNK * 128)],
        )


@functools.partial(
    pl.kernel,
    out_type=jax.ShapeDtypeStruct((N_TOK * 128,), jnp.float32),
    mesh=plsc.VectorSubcoreMesh(core_axis_name="c", subcore_axis_name="s"),
    compiler_params=pltpu.CompilerParams(needs_layout_passes=False),
    scratch_types=[
        pltpu.VMEM((32 * L,), jnp.float32),
        pltpu.VMEM((V * L,), jnp.float32),
        pltpu.VMEM((TOK_PER_W,), jnp.int32),
        pltpu.VMEM((CHUNK * 128,), jnp.float32),
    ],
)
def _sc_lookup(ids_hbm, par_hbm, out_hbm, *scratch):
    _body(ids_hbm, par_hbm, out_hbm, *scratch)


def kernel(input_ids, emb, W1, b1, Wh, bh):
    ids = input_ids.reshape(-1).astype(jnp.int32)

    def pad16(x):
        return jnp.pad(x, ((0, 0), (0, L - D)))

    par = jnp.concatenate(
        [
            pad16(emb),
            pad16(W1.T),
            pad16(Wh.T),
            jnp.pad(b1, (0, L - D))[None, :],
            jnp.pad(bh, (0, L - D))[None, :],
        ],
        axis=0,
    ).reshape(-1)
    out = _sc_lookup(ids, par)
    return out.reshape(4, 8192, 128)[..., :D]


# speedup vs baseline: 2.6893x; 1.0186x over previous
import functools

import jax
import jax.numpy as jnp
import numpy as np
from jax import lax
from jax.experimental import pallas as pl
from jax.experimental.pallas import tpu as pltpu
from jax.experimental.pallas import tpu_sc as plsc

NC = 2
NS = 16
NW = NC * NS
L = 16

V = 10
D = 10
N_TOK = 4 * 8192
TOK_PER_W = N_TOK // NW
CHUNK = 512

_GATHER_DNUMS = lax.GatherDimensionNumbers(
    offset_dims=(), collapsed_slice_dims=(0,), start_index_map=(0,))


def _reg_gather(x, idx16):
    return lax.gather(x, idx16.reshape(L, 1), _GATHER_DNUMS, (1,),
                      mode=lax.GatherScatterMode.PROMISE_IN_BOUNDS)


def _splat(x, k):
    return _reg_gather(x, jnp.full((L,), k, jnp.int32))


def _body(ids_hbm, emb_hbm, w1_hbm, b1_hbm, wh_hbm, bh_hbm, out_hbm,
          par_v, table_v, ids_v, out_v, sem):
    wid = lax.axis_index("s") * NC + lax.axis_index("c")
    iota16 = lax.iota(jnp.int32, L)

    cps = [
        pltpu.async_copy(emb_hbm, par_v.at[pl.ds(0, 100)], sem),
        pltpu.async_copy(w1_hbm, par_v.at[pl.ds(112, 100)], sem),
        pltpu.async_copy(wh_hbm, par_v.at[pl.ds(224, 100)], sem),
        pltpu.async_copy(b1_hbm, par_v.at[pl.ds(336, 10)], sem),
        pltpu.async_copy(bh_hbm, par_v.at[pl.ds(352, 10)], sem),
    ]
    pltpu.sync_copy(ids_hbm.at[pl.ds(wid * TOK_PER_W, TOK_PER_W)], ids_v)
    for cp in cps:
        cp.wait()

    def col(base, k):
        return plsc.load_gather(par_v, [base + k + 10 * iota16])

    b1v = plsc.load_gather(par_v, [336 + iota16])
    bhv = plsc.load_gather(par_v, [352 + iota16])

    for v in range(V):
        e = plsc.load_gather(par_v, [10 * v + iota16])
        h = b1v
        for k in range(D):
            h = h + _splat(e, k) * col(112, k)
        t = bhv
        for k in range(D):
            t = t + _splat(h, k) * col(224, k)
        table_v[pl.ds(v * L, L)] = t

    for c in range(TOK_PER_W // CHUNK):

        def blk(b, _):
            sid16 = ids_v[pl.ds(c * CHUNK + b * L, L)] << 4
            for j in range(L):
                idx = _reg_gather(sid16, jnp.full((L,), j, jnp.int32)) + iota16
                out_v[pl.ds((b * L + j) * 128, L)] = plsc.load_gather(
                    table_v, [idx])
            return 0

        lax.fori_loop(0, CHUNK // L, blk, 0)
        pltpu.sync_copy(
            out_v,
            out_hbm.at[pl.ds((wid * TOK_PER_W + c * CHUNK) * 128,
                             CHUNK * 128)],
        )


@functools.partial(
    pl.kernel,
    out_type=jax.ShapeDtypeStruct((N_TOK * 128,), jnp.float32),
    mesh=plsc.VectorSubcoreMesh(core_axis_name="c", subcore_axis_name="s"),
    compiler_params=pltpu.CompilerParams(needs_layout_passes=False),
    scratch_types=[
        pltpu.VMEM((512,), jnp.float32),
        pltpu.VMEM((V * L,), jnp.float32),
        pltpu.VMEM((TOK_PER_W,), jnp.int32),
        pltpu.VMEM((CHUNK * 128,), jnp.float32),
        pltpu.SemaphoreType.DMA,
    ],
)
def _sc_lookup(ids_hbm, emb_hbm, w1_hbm, b1_hbm, wh_hbm, bh_hbm, out_hbm,
               *scratch):
    _body(ids_hbm, emb_hbm, w1_hbm, b1_hbm, wh_hbm, bh_hbm, out_hbm,
          *scratch)


def kernel(input_ids, emb, W1, b1, Wh, bh):
    ids = input_ids.reshape(-1).astype(jnp.int32)
    out = _sc_lookup(ids, emb.reshape(-1), W1.reshape(-1), b1,
                     Wh.reshape(-1), bh)
    return out.reshape(4, 8192, 128)[..., :D]


# speedup vs baseline: 2.8747x; 1.0689x over previous
import functools

import jax
import jax.numpy as jnp
import numpy as np
from jax import lax
from jax.experimental import pallas as pl
from jax.experimental.pallas import tpu as pltpu
from jax.experimental.pallas import tpu_sc as plsc

NC = 2
NS = 16
NW = NC * NS
L = 16

V = 10
D = 10
N_TOK = 4 * 8192
TOK_PER_W = N_TOK // NW
CHUNK = 256

_GATHER_DNUMS = lax.GatherDimensionNumbers(
    offset_dims=(), collapsed_slice_dims=(0,), start_index_map=(0,))


def _reg_gather(x, idx16):
    return lax.gather(x, idx16.reshape(L, 1), _GATHER_DNUMS, (1,),
                      mode=lax.GatherScatterMode.PROMISE_IN_BOUNDS)


def _splat(x, k):
    return _reg_gather(x, jnp.full((L,), k, jnp.int32))


def _body(ids_hbm, emb_hbm, w1_hbm, b1_hbm, wh_hbm, bh_hbm, out_hbm,
          par_v, table_v, ids_v, out_a, out_b, sem, sem_a, sem_b):
    wid = lax.axis_index("s") * NC + lax.axis_index("c")
    iota16 = lax.iota(jnp.int32, L)

    cps = [
        pltpu.async_copy(emb_hbm, par_v.at[pl.ds(0, 100)], sem),
        pltpu.async_copy(w1_hbm, par_v.at[pl.ds(112, 100)], sem),
        pltpu.async_copy(wh_hbm, par_v.at[pl.ds(224, 100)], sem),
        pltpu.async_copy(b1_hbm, par_v.at[pl.ds(336, 10)], sem),
        pltpu.async_copy(bh_hbm, par_v.at[pl.ds(352, 10)], sem),
    ]
    pltpu.sync_copy(ids_hbm.at[pl.ds(wid * TOK_PER_W, TOK_PER_W)], ids_v)
    for cp in cps:
        cp.wait()

    def col(base, k):
        return plsc.load_gather(par_v, [base + k + 10 * iota16])

    b1v = plsc.load_gather(par_v, [336 + iota16])
    bhv = plsc.load_gather(par_v, [352 + iota16])

    for v in range(V):
        e = plsc.load_gather(par_v, [10 * v + iota16])
        h = b1v
        for k in range(D):
            h = h + _splat(e, k) * col(112, k)
        t = bhv
        for k in range(D):
            t = t + _splat(h, k) * col(224, k)
        table_v[pl.ds(v * L, L)] = t

    bufs = (out_a, out_b)
    sems = (sem_a, sem_b)
    descs = [None, None]
    for c in range(TOK_PER_W // CHUNK):
        buf = bufs[c & 1]
        if descs[c & 1] is not None:
            descs[c & 1].wait()

        @plsc.parallel_loop(0, CHUNK // L, unroll=2)
        def blk(b):
            sid16 = ids_v[pl.ds(c * CHUNK + b * L, L)] << 4
            for j in range(L):
                idx = _reg_gather(sid16, jnp.full((L,), j, jnp.int32)) + iota16
                buf[pl.ds((b * L + j) * 128, L)] = plsc.load_gather(
                    table_v, [idx])

        descs[c & 1] = pltpu.make_async_copy(
            buf,
            out_hbm.at[pl.ds((wid * TOK_PER_W + c * CHUNK) * 128,
                             CHUNK * 128)],
            sems[c & 1],
        )
        descs[c & 1].start()
    for d in descs:
        d.wait()


@functools.partial(
    pl.kernel,
    out_type=jax.ShapeDtypeStruct((N_TOK * 128,), jnp.float32),
    mesh=plsc.VectorSubcoreMesh(core_axis_name="c", subcore_axis_name="s"),
    compiler_params=pltpu.CompilerParams(needs_layout_passes=False),
    scratch_types=[
        pltpu.VMEM((512,), jnp.float32),
        pltpu.VMEM((V * L,), jnp.float32),
        pltpu.VMEM((TOK_PER_W,), jnp.int32),
        pltpu.VMEM((CHUNK * 128,), jnp.float32),
        pltpu.VMEM((CHUNK * 128,), jnp.float32),
        pltpu.SemaphoreType.DMA,
        pltpu.SemaphoreType.DMA,
        pltpu.SemaphoreType.DMA,
    ],
)
def _sc_lookup(ids_hbm, emb_hbm, w1_hbm, b1_hbm, wh_hbm, bh_hbm, out_hbm,
               *scratch):
    _body(ids_hbm, emb_hbm, w1_hbm, b1_hbm, wh_hbm, bh_hbm, out_hbm,
          *scratch)


def kernel(input_ids, emb, W1, b1, Wh, bh):
    ids = input_ids.reshape(-1).astype(jnp.int32)
    out = _sc_lookup(ids, emb.reshape(-1), W1.reshape(-1), b1,
                     Wh.reshape(-1), bh)
    return out.reshape(4, 8192, 128)[..., :D]


# speedup vs baseline: 2.8889x; 1.0049x over previous
import functools

import jax
import jax.numpy as jnp
import numpy as np
from jax import lax
from jax.experimental import pallas as pl
from jax.experimental.pallas import tpu as pltpu
from jax.experimental.pallas import tpu_sc as plsc

NC = 2
NS = 16
NW = NC * NS
L = 16

V = 10
D = 10
N_TOK = 4 * 8192
TOK_PER_W = N_TOK // NW
CHUNK = 256

_GATHER_DNUMS = lax.GatherDimensionNumbers(
    offset_dims=(), collapsed_slice_dims=(0,), start_index_map=(0,))


def _reg_gather(x, idx16):
    return lax.gather(x, idx16.reshape(L, 1), _GATHER_DNUMS, (1,),
                      mode=lax.GatherScatterMode.PROMISE_IN_BOUNDS)


def _splat(x, k):
    return _reg_gather(x, jnp.full((L,), k, jnp.int32))


def _body(ids_hbm, par_hbm, out_hbm,
          par_v, table_v, ids_v, out_a, out_b, sem, sem_a, sem_b):
    wid = lax.axis_index("s") * NC + lax.axis_index("c")
    iota16 = lax.iota(jnp.int32, L)

    cp = pltpu.async_copy(par_hbm, par_v.at[pl.ds(0, 320)], sem)
    pltpu.sync_copy(ids_hbm.at[pl.ds(wid * TOK_PER_W, TOK_PER_W)], ids_v)
    cp.wait()

    def col(base, k):
        return plsc.load_gather(par_v, [base + k + 10 * iota16])

    b1v = plsc.load_gather(par_v, [300 + iota16])
    bhv = plsc.load_gather(par_v, [310 + iota16])

    for v in range(V):
        e = plsc.load_gather(par_v, [10 * v + iota16])
        h = b1v
        for k in range(D):
            h = h + _splat(e, k) * col(100, k)
        t = bhv
        for k in range(D):
            t = t + _splat(h, k) * col(200, k)
        table_v[pl.ds(v * L, L)] = t

    bufs = (out_a, out_b)
    sems = (sem_a, sem_b)
    descs = [None, None]
    for c in range(TOK_PER_W // CHUNK):
        buf = bufs[c & 1]
        if descs[c & 1] is not None:
            descs[c & 1].wait()

        @plsc.parallel_loop(0, CHUNK // L, unroll=2)
        def blk(b):
            sid16 = ids_v[pl.ds(c * CHUNK + b * L, L)] << 4
            for j in range(L):
                idx = _reg_gather(sid16, jnp.full((L,), j, jnp.int32)) + iota16
                buf[pl.ds((b * L + j) * 128, L)] = plsc.load_gather(
                    table_v, [idx])

        descs[c & 1] = pltpu.make_async_copy(
            buf,
            out_hbm.at[pl.ds((wid * TOK_PER_W + c * CHUNK) * 128,
                             CHUNK * 128)],
            sems[c & 1],
        )
        descs[c & 1].start()
    for d in descs:
        d.wait()


@functools.partial(
    pl.kernel,
    out_type=jax.ShapeDtypeStruct((N_TOK * 128,), jnp.float32),
    mesh=plsc.VectorSubcoreMesh(core_axis_name="c", subcore_axis_name="s"),
    compiler_params=pltpu.CompilerParams(needs_layout_passes=False),
    scratch_types=[
        pltpu.VMEM((512,), jnp.float32),
        pltpu.VMEM((V * L,), jnp.float32),
        pltpu.VMEM((TOK_PER_W,), jnp.int32),
        pltpu.VMEM((CHUNK * 128,), jnp.float32),
        pltpu.VMEM((CHUNK * 128,), jnp.float32),
        pltpu.SemaphoreType.DMA,
        pltpu.SemaphoreType.DMA,
        pltpu.SemaphoreType.DMA,
    ],
)
def _sc_lookup(ids_hbm, par_hbm, out_hbm, *scratch):
    _body(ids_hbm, par_hbm, out_hbm, *scratch)


def kernel(input_ids, emb, W1, b1, Wh, bh):
    ids = input_ids.reshape(-1).astype(jnp.int32)
    par = jnp.concatenate(
        [emb.reshape(-1), W1.reshape(-1), Wh.reshape(-1), b1, bh])
    out = _sc_lookup(ids, par)
    return out.reshape(4, 8192, 128)[..., :D]
